# baseline (speedup 1.0000x reference)
import jax
import jax.numpy as jnp
from jax.experimental import pallas as pl
from jax.experimental.pallas import tpu as pltpu

N = 10000
K = 32
H = 128
NT = 4
BLOCK_N = 200


def _tree_cell_kernel(oneh_ref, s_ref, nh_ref, nc_ref, fin_ref,
                      uiou_ref, biou_ref, ufh_ref, bf_ref,
                      iou_out, c_out):
    nh = nh_ref[...].astype(jnp.bfloat16)
    oneh = oneh_ref[...]
    oneh_b = oneh.astype(jnp.bfloat16)

    x_wide = jnp.concatenate(
        [nh * oneh_b[:, t][:, None, None] for t in range(NT)],
        axis=-1).reshape(BLOCK_N * K, NT * H)

    fpre_half = jnp.dot(x_wide, ufh_ref[...],
                        preferred_element_type=jnp.float32)
    fpre_half = fpre_half.reshape(BLOCK_N, K, H)

    nh2 = nh.reshape(BLOCK_N * K, H)
    h_tilde = jnp.dot(s_ref[...], nh2,
                      preferred_element_type=jnp.float32)

    ht_wide = jnp.concatenate(
        [h_tilde * oneh[:, t][:, None] for t in range(NT)],
        axis=-1).astype(jnp.bfloat16)
    iou = jnp.dot(ht_wide, uiou_ref[...],
                  preferred_element_type=jnp.float32)
    b_iou_sel = jnp.dot(oneh, biou_ref[...],
                        preferred_element_type=jnp.float32)
    iou_out[...] = iou + b_iou_sel

    b_f_sel = jnp.dot(oneh, bf_ref[...],
                      preferred_element_type=jnp.float32)
    hb = 0.5 * (fin_ref[...] + b_f_sel)
    f = 0.5 * jnp.tanh(fpre_half + hb[:, None, :]) + 0.5
    c_out[...] = jnp.sum(f * nc_ref[...], axis=1)


@jax.jit
def kernel(n_h, n_c, f_in, type_id, U_iou, b_iou, U_f, b_f):
    tid = type_id.astype(jnp.int32).reshape(N, 1)
    oneh = (tid == jnp.arange(NT, dtype=jnp.int32)[None, :]).astype(jnp.float32)
    uf_half = (U_f * 0.5).reshape(NT * H, H).astype(jnp.bfloat16)
    uiou_flat = U_iou.reshape(NT * H, 3 * H).astype(jnp.bfloat16)
    s_mat = jnp.kron(jnp.eye(BLOCK_N, dtype=jnp.bfloat16),
                     jnp.ones((1, K), dtype=jnp.bfloat16))

    grid = (N // BLOCK_N,)
    out = pl.pallas_call(
        _tree_cell_kernel,
        grid=grid,
        in_specs=[
            pl.BlockSpec((BLOCK_N, NT), lambda i: (i, 0)),
            pl.BlockSpec((BLOCK_N, BLOCK_N * K), lambda i: (0, 0)),
            pl.BlockSpec((BLOCK_N, K, H), lambda i: (i, 0, 0)),
            pl.BlockSpec((BLOCK_N, K, H), lambda i: (i, 0, 0)),
            pl.BlockSpec((BLOCK_N, H), lambda i: (i, 0)),
            pl.BlockSpec((NT * H, 3 * H), lambda i: (0, 0)),
            pl.BlockSpec((NT, 3 * H), lambda i: (0, 0)),
            pl.BlockSpec((NT * H, H), lambda i: (0, 0)),
            pl.BlockSpec((NT, H), lambda i: (0, 0)),
        ],
        out_specs=[
            pl.BlockSpec((BLOCK_N, 3 * H), lambda i: (i, 0)),
            pl.BlockSpec((BLOCK_N, H), lambda i: (i, 0)),
        ],
        out_shape=[
            jax.ShapeDtypeStruct((N, 3 * H), jnp.float32),
            jax.ShapeDtypeStruct((N, H), jnp.float32),
        ],
        compiler_params=pltpu.CompilerParams(
            dimension_semantics=("arbitrary",),
        ),
    )(oneh, s_mat, n_h, n_c, f_in, uiou_flat, b_iou, uf_half, b_f)
    return out[0], out[1]


# speedup vs baseline: 1.3103x; 1.3103x over previous
import jax
import jax.numpy as jnp
from jax.experimental import pallas as pl
from jax.experimental.pallas import tpu as pltpu

N = 10000
K = 32
H = 128
NT = 4
BLOCK_N = 200


def _tree_cell_kernel(oneh_ref, tid_ref, nh_ref, nc_ref, fin_ref,
                      uiou_ref, biou_ref, uf_ref, bf_ref,
                      iou_out, c_out):
    nh = nh_ref[...]
    oneh = oneh_ref[...]
    tid = tid_ref[...]
    h_tilde = jnp.sum(nh, axis=1)
    nh2 = nh.reshape(BLOCK_N * K, H).astype(jnp.bfloat16)

    b_iou_sel = jnp.dot(oneh, biou_ref[...],
                        preferred_element_type=jnp.float32)
    b_f_sel = jnp.dot(oneh, bf_ref[...],
                      preferred_element_type=jnp.float32)

    iou_t = [jnp.dot(h_tilde, uiou_ref[t], preferred_element_type=jnp.float32)
             for t in range(NT)]
    f_t = [jnp.dot(nh2, uf_ref[t],
                   preferred_element_type=jnp.float32).reshape(BLOCK_N, K, H)
           for t in range(NT)]

    bit0 = (tid & 1) == 1
    bit1 = (tid & 2) == 2
    iou = jnp.where(bit1,
                    jnp.where(bit0, iou_t[3], iou_t[2]),
                    jnp.where(bit0, iou_t[1], iou_t[0]))
    b0 = bit0[:, :, None]
    b1 = bit1[:, :, None]
    fpre = jnp.where(b1,
                     jnp.where(b0, f_t[3], f_t[2]),
                     jnp.where(b0, f_t[1], f_t[0]))

    hb = 0.5 * (fin_ref[...] + b_f_sel)
    f = 0.5 * jnp.tanh(fpre + hb[:, None, :]) + 0.5
    c_out[...] = jnp.sum(f * nc_ref[...], axis=1)
    iou_out[...] = iou + b_iou_sel


@jax.jit
def kernel(n_h, n_c, f_in, type_id, U_iou, b_iou, U_f, b_f):
    tid = type_id.astype(jnp.int32).reshape(N, 1)
    oneh = (tid == jnp.arange(NT, dtype=jnp.int32)[None, :]).astype(jnp.float32)
    uf_half = (U_f * 0.5).astype(jnp.bfloat16)

    grid = (N // BLOCK_N,)
    out = pl.pallas_call(
        _tree_cell_kernel,
        grid=grid,
        in_specs=[
            pl.BlockSpec((BLOCK_N, NT), lambda i: (i, 0)),
            pl.BlockSpec((BLOCK_N, 1), lambda i: (i, 0)),
            pl.BlockSpec((BLOCK_N, K, H), lambda i: (i, 0, 0)),
            pl.BlockSpec((BLOCK_N, K, H), lambda i: (i, 0, 0)),
            pl.BlockSpec((BLOCK_N, H), lambda i: (i, 0)),
            pl.BlockSpec((NT, H, 3 * H), lambda i: (0, 0, 0)),
            pl.BlockSpec((NT, 3 * H), lambda i: (0, 0)),
            pl.BlockSpec((NT, H, H), lambda i: (0, 0, 0)),
            pl.BlockSpec((NT, H), lambda i: (0, 0)),
        ],
        out_specs=[
            pl.BlockSpec((BLOCK_N, 3 * H), lambda i: (i, 0)),
            pl.BlockSpec((BLOCK_N, H), lambda i: (i, 0)),
        ],
        out_shape=[
            jax.ShapeDtypeStruct((N, 3 * H), jnp.float32),
            jax.ShapeDtypeStruct((N, H), jnp.float32),
        ],
        compiler_params=pltpu.CompilerParams(
            dimension_semantics=("arbitrary",),
        ),
    )(oneh, tid, n_h, n_c, f_in, U_iou, b_iou, uf_half, b_f)
    return out[0], out[1]
